# Initial kernel scaffold; baseline (speedup 1.0000x reference)
import jax
import jax.numpy as jnp
from jax.experimental import pallas as pl


def kernel(x, edge_index, W1, b1, W2, b2, W3, b3, Wl, bl):
    raise NotImplementedError("write your pallas kernel here")


# speedup vs baseline: 8.8700x; 8.8700x over previous
import functools

import jax
import jax.numpy as jnp
from jax import lax
from jax.experimental import pallas as pl
from jax.experimental.pallas import tpu as pltpu
from jax.experimental.pallas import tpu_sc as plsc

N = 10000
E = 160000
F = 256
HF = 128
BLK = 1000
CH = 128
NCH = E // CH
NSUB = 16
RPS = 624
TAIL = N - RPS * NSUB

_f32 = jnp.float32
_mesh = plsc.VectorSubcoreMesh(core_axis_name="c", subcore_axis_name="s")


@functools.partial(
    pl.kernel,
    out_type=(jax.ShapeDtypeStruct((N, 16), _f32),
              jax.ShapeDtypeStruct((N, 16), _f32)),
    mesh=_mesh,
    scratch_types=[
        pltpu.VMEM((CH,), jnp.int32),
        pltpu.VMEM((CH, 16), _f32),
        pltpu.VMEM_SHARED((N, 16), _f32),
    ],
)
def _sc_degree(dst_hbm, ones_hbm, degA_hbm, degB_hbm, dstbuf, onesbuf, acc):
    c = lax.axis_index("c")
    s = lax.axis_index("s")

    def rowcopy(from_ref, to_ref):
        sl = pl.ds(s * RPS, RPS)
        pltpu.sync_copy(from_ref.at[sl], to_ref.at[sl])

        @pl.when(s == NSUB - 1)
        def _():
            tl = pl.ds(RPS * NSUB, TAIL)
            pltpu.sync_copy(from_ref.at[tl], to_ref.at[tl])

    rowcopy(ones_hbm, acc)
    pltpu.sync_copy(ones_hbm.at[pl.ds(0, CH)], onesbuf)
    plsc.subcore_barrier()
    w = c * NSUB + s
    lo = (w * NCH) // (2 * NSUB)
    hi = ((w + 1) * NCH) // (2 * NSUB)

    def chunk(j, carry):
        pltpu.sync_copy(dst_hbm.at[pl.ds(j * CH, CH)], dstbuf)
        pltpu.sync_copy(onesbuf, acc.at[dstbuf], add=True)
        return carry

    lax.fori_loop(lo, hi, chunk, 0)
    plsc.subcore_barrier()

    @pl.when(c == 0)
    def _():
        rowcopy(acc, degA_hbm)

    @pl.when(c == 1)
    def _():
        rowcopy(acc, degB_hbm)


@functools.partial(
    pl.kernel,
    out_type=(jax.ShapeDtypeStruct((N, HF), _f32),
              jax.ShapeDtypeStruct((N, HF), _f32)),
    mesh=_mesh,
    scratch_types=[
        pltpu.VMEM((CH,), jnp.int32),
        pltpu.VMEM((CH,), jnp.int32),
        pltpu.VMEM((CH, HF), _f32),
        pltpu.VMEM_SHARED((N, HF), _f32),
        pltpu.SemaphoreType.DMA,
    ],
)
def _sc_agg(src_hbm, dst_hbm, hL_hbm, hR_hbm, aggL_hbm, aggR_hbm,
            srcbuf, dstbuf, rowbuf, acc, sem):
    c = lax.axis_index("c")
    s = lax.axis_index("s")
    lo = (s * NCH) // NSUB
    hi = ((s + 1) * NCH) // NSUB

    def rowcopy(from_ref, to_ref):
        sl = pl.ds(s * RPS, RPS)
        pltpu.sync_copy(from_ref.at[sl], to_ref.at[sl])

        @pl.when(s == NSUB - 1)
        def _():
            tl = pl.ds(RPS * NSUB, TAIL)
            pltpu.sync_copy(from_ref.at[tl], to_ref.at[tl])

    def run(h_hbm, out_hbm):
        rowcopy(h_hbm, acc)
        plsc.subcore_barrier()

        def chunk(j, carry):
            pltpu.sync_copy(src_hbm.at[pl.ds(j * CH, CH)], srcbuf)
            pltpu.sync_copy(dst_hbm.at[pl.ds(j * CH, CH)], dstbuf)
            pltpu.async_copy(h_hbm.at[srcbuf], rowbuf, sem).wait()
            pltpu.sync_copy(rowbuf, acc.at[dstbuf], add=True)
            return carry

        lax.fori_loop(lo, hi, chunk, 0)
        plsc.subcore_barrier()
        rowcopy(acc, out_hbm)

    @pl.when(c == 0)
    def _():
        run(hL_hbm, aggL_hbm)

    @pl.when(c == 1)
    def _():
        run(hR_hbm, aggR_hbm)


def _dot(a, b):
    return jnp.dot(a, b, preferred_element_type=_f32,
                   precision=lax.Precision.HIGHEST)


def _tc1_body(x_ref, w_ref, dA_ref, dB_ref, hL_ref, hR_ref, dinv_ref):
    deg = dA_ref[:, 0:1] + dB_ref[:, 0:1] - 1.0
    dinv = lax.rsqrt(deg)
    h = _dot(x_ref[...], w_ref[...]) * dinv
    hL_ref[...] = h[:, :HF]
    hR_ref[...] = h[:, HF:]
    dinv_ref[...] = dinv


def _tc1(x, W1, degA, degB):
    return pl.pallas_call(
        _tc1_body,
        grid=(N // BLK,),
        in_specs=[
            pl.BlockSpec((BLK, F), lambda i: (i, 0)),
            pl.BlockSpec((F, F), lambda i: (0, 0)),
            pl.BlockSpec((BLK, 16), lambda i: (i, 0)),
            pl.BlockSpec((BLK, 16), lambda i: (i, 0)),
        ],
        out_specs=[
            pl.BlockSpec((BLK, HF), lambda i: (i, 0)),
            pl.BlockSpec((BLK, HF), lambda i: (i, 0)),
            pl.BlockSpec((BLK, 1), lambda i: (i, 0)),
        ],
        out_shape=[
            jax.ShapeDtypeStruct((N, HF), _f32),
            jax.ShapeDtypeStruct((N, HF), _f32),
            jax.ShapeDtypeStruct((N, 1), _f32),
        ],
    )(x, W1, degA, degB)


def _tcl_body(aL_ref, aR_ref, dinv_ref, b_ref, w_ref, hL_ref, hR_ref):
    dinv = dinv_ref[...]
    agg = jnp.concatenate([aL_ref[...], aR_ref[...]], axis=1)
    t = jnp.maximum(agg * dinv + b_ref[...], 0.0)
    h = _dot(t, w_ref[...]) * dinv
    hL_ref[...] = h[:, :HF]
    hR_ref[...] = h[:, HF:]


def _tcl(aL, aR, dinv, b, W):
    return pl.pallas_call(
        _tcl_body,
        grid=(N // BLK,),
        in_specs=[
            pl.BlockSpec((BLK, HF), lambda i: (i, 0)),
            pl.BlockSpec((BLK, HF), lambda i: (i, 0)),
            pl.BlockSpec((BLK, 1), lambda i: (i, 0)),
            pl.BlockSpec((1, F), lambda i: (0, 0)),
            pl.BlockSpec((F, F), lambda i: (0, 0)),
        ],
        out_specs=[
            pl.BlockSpec((BLK, HF), lambda i: (i, 0)),
            pl.BlockSpec((BLK, HF), lambda i: (i, 0)),
        ],
        out_shape=[
            jax.ShapeDtypeStruct((N, HF), _f32),
            jax.ShapeDtypeStruct((N, HF), _f32),
        ],
    )(aL, aR, dinv, b, W)


def _tcf_body(aL_ref, aR_ref, dinv_ref, b_ref, wl_ref, bl_ref, y_ref):
    dinv = dinv_ref[...]
    agg = jnp.concatenate([aL_ref[...], aR_ref[...]], axis=1)
    t = jnp.maximum(agg * dinv + b_ref[...], 0.0)
    y = _dot(t, wl_ref[...]) + bl_ref[...]
    y_ref[...] = jax.nn.sigmoid(y)


def _tcf(aL, aR, dinv, b3, Wl, bl):
    return pl.pallas_call(
        _tcf_body,
        grid=(N // BLK,),
        in_specs=[
            pl.BlockSpec((BLK, HF), lambda i: (i, 0)),
            pl.BlockSpec((BLK, HF), lambda i: (i, 0)),
            pl.BlockSpec((BLK, 1), lambda i: (i, 0)),
            pl.BlockSpec((1, F), lambda i: (0, 0)),
            pl.BlockSpec((F, 1), lambda i: (0, 0)),
            pl.BlockSpec((1, 1), lambda i: (0, 0)),
        ],
        out_specs=[pl.BlockSpec((BLK, 1), lambda i: (i, 0))],
        out_shape=[jax.ShapeDtypeStruct((N, 1), _f32)],
    )(aL, aR, dinv, b3, Wl, bl)


def kernel(x, edge_index, W1, b1, W2, b2, W3, b3, Wl, bl):
    ones16 = jnp.ones((N, 16), _f32)
    src, dst = edge_index[0], edge_index[1]
    degA, degB = _sc_degree(dst, ones16)
    hL, hR, dinv = _tc1(x, W1, degA, degB)
    aL, aR = _sc_agg(src, dst, hL, hR)
    hL, hR = _tcl(aL, aR, dinv, b1.reshape(1, F), W2)
    aL, aR = _sc_agg(src, dst, hL, hR)
    hL, hR = _tcl(aL, aR, dinv, b2.reshape(1, F), W3)
    aL, aR = _sc_agg(src, dst, hL, hR)
    (y,) = _tcf(aL, aR, dinv, b3.reshape(1, F), Wl, bl.reshape(1, 1))
    return y.reshape(-1)


# speedup vs baseline: 14.6712x; 1.6540x over previous
import functools

import jax
import jax.numpy as jnp
from jax import lax
from jax.experimental import pallas as pl
from jax.experimental.pallas import tpu as pltpu
from jax.experimental.pallas import tpu_sc as plsc

N = 10000
E = 160000
F = 256
HF = 128
BLK = 1000
CH = 128
NCH = E // CH
NSUB = 16
RPS = 624
TAIL = N - RPS * NSUB

_f32 = jnp.float32
_mesh = plsc.VectorSubcoreMesh(core_axis_name="c", subcore_axis_name="s")


@functools.partial(
    pl.kernel,
    out_type=(jax.ShapeDtypeStruct((N, 16), _f32),
              jax.ShapeDtypeStruct((N, 16), _f32)),
    mesh=_mesh,
    scratch_types=[
        pltpu.VMEM((CH,), jnp.int32),
        pltpu.VMEM((CH, 16), _f32),
        pltpu.VMEM_SHARED((N, 16), _f32),
    ],
)
def _sc_degree(dst_hbm, ones_hbm, degA_hbm, degB_hbm, dstbuf, onesbuf, acc):
    c = lax.axis_index("c")
    s = lax.axis_index("s")

    def rowcopy(from_ref, to_ref):
        sl = pl.ds(s * RPS, RPS)
        pltpu.sync_copy(from_ref.at[sl], to_ref.at[sl])

        @pl.when(s == NSUB - 1)
        def _():
            tl = pl.ds(RPS * NSUB, TAIL)
            pltpu.sync_copy(from_ref.at[tl], to_ref.at[tl])

    rowcopy(ones_hbm, acc)
    pltpu.sync_copy(ones_hbm.at[pl.ds(0, CH)], onesbuf)
    plsc.subcore_barrier()
    w = c * NSUB + s
    lo = (w * NCH) // (2 * NSUB)
    hi = ((w + 1) * NCH) // (2 * NSUB)

    def chunk(j, carry):
        pltpu.sync_copy(dst_hbm.at[pl.ds(j * CH, CH)], dstbuf)
        pltpu.sync_copy(onesbuf, acc.at[dstbuf], add=True)
        return carry

    lax.fori_loop(lo, hi, chunk, 0)
    plsc.subcore_barrier()

    @pl.when(c == 0)
    def _():
        rowcopy(acc, degA_hbm)

    @pl.when(c == 1)
    def _():
        rowcopy(acc, degB_hbm)


EPS = E // NSUB
NCHS = EPS // CH
REM = EPS - NCHS * CH


@functools.partial(
    pl.kernel,
    out_type=(jax.ShapeDtypeStruct((N, HF), _f32),
              jax.ShapeDtypeStruct((N, HF), _f32)),
    mesh=_mesh,
    scratch_types=[
        pltpu.VMEM((EPS,), jnp.int32),
        pltpu.VMEM((2, CH), jnp.int32),
        pltpu.VMEM((1, REM), jnp.int32),
        pltpu.VMEM((2, CH, HF), _f32),
        pltpu.VMEM((REM, HF), _f32),
        pltpu.VMEM_SHARED((N, HF), _f32),
        pltpu.SemaphoreType.DMA,
        pltpu.SemaphoreType.DMA,
        pltpu.SemaphoreType.DMA,
        pltpu.SemaphoreType.DMA,
    ],
)
def _sc_agg(src_hbm, dst_hbm, hL_hbm, hR_hbm, aggL_hbm, aggR_hbm,
            srcslab, dstbuf, dstbuf16, rowbuf, rowbuf16, acc,
            semd0, semd1, semg0, semg1):
    c = lax.axis_index("c")
    s = lax.axis_index("s")
    semd = (semd0, semd1)
    semg = (semg0, semg1)

    def rowcopy(from_ref, to_ref):
        sl = pl.ds(s * RPS, RPS)
        pltpu.sync_copy(from_ref.at[sl], to_ref.at[sl])

        @pl.when(s == NSUB - 1)
        def _():
            tl = pl.ds(RPS * NSUB, TAIL)
            pltpu.sync_copy(from_ref.at[tl], to_ref.at[tl])

    def run(h_hbm, out_hbm):
        pltpu.sync_copy(src_hbm.at[pl.ds(s * EPS, EPS)], srcslab)
        rowcopy(h_hbm, acc)

        def dst_dma(j, b):
            return pltpu.make_async_copy(
                dst_hbm.at[pl.ds(s * EPS + j * CH, CH)], dstbuf.at[b], semd[b])

        def gat_dma(j, b):
            return pltpu.make_async_copy(
                h_hbm.at[srcslab.at[pl.ds(j * CH, CH)]], rowbuf.at[b], semg[b])

        gat_dma(0, 0).start()
        dst_dma(0, 0).start()
        plsc.subcore_barrier()

        def pair(k, carry):
            for b in (0, 1):
                j = 2 * k + b
                gat_dma(j, b).wait()
                dst_dma(j, b).wait()

                @pl.when(j + 1 < NCHS)
                def _():
                    gat_dma(j + 1, 1 - b).start()

                pltpu.sync_copy(rowbuf.at[b], acc.at[dstbuf.at[b]], add=True)

                @pl.when(j + 1 < NCHS)
                def _():
                    dst_dma(j + 1, 1 - b).start()

            return carry

        lax.fori_loop(0, NCHS // 2, pair, 0)
        pltpu.sync_copy(dst_hbm.at[pl.ds(s * EPS + NCHS * CH, REM)],
                        dstbuf16.at[0])
        pltpu.async_copy(h_hbm.at[srcslab.at[pl.ds(NCHS * CH, REM)]],
                         rowbuf16, semg0).wait()
        pltpu.sync_copy(rowbuf16, acc.at[dstbuf16.at[0]], add=True)
        plsc.subcore_barrier()
        rowcopy(acc, out_hbm)

    @pl.when(c == 0)
    def _():
        run(hL_hbm, aggL_hbm)

    @pl.when(c == 1)
    def _():
        run(hR_hbm, aggR_hbm)


def _dot(a, b):
    return jnp.dot(a, b, preferred_element_type=_f32,
                   precision=lax.Precision.HIGHEST)


def _tc1_body(x_ref, w_ref, dA_ref, dB_ref, hL_ref, hR_ref, dinv_ref):
    deg = dA_ref[:, 0:1] + dB_ref[:, 0:1] - 1.0
    dinv = lax.rsqrt(deg)
    h = _dot(x_ref[...], w_ref[...]) * dinv
    hL_ref[...] = h[:, :HF]
    hR_ref[...] = h[:, HF:]
    dinv_ref[...] = dinv


def _tc1(x, W1, degA, degB):
    return pl.pallas_call(
        _tc1_body,
        grid=(N // BLK,),
        in_specs=[
            pl.BlockSpec((BLK, F), lambda i: (i, 0)),
            pl.BlockSpec((F, F), lambda i: (0, 0)),
            pl.BlockSpec((BLK, 16), lambda i: (i, 0)),
            pl.BlockSpec((BLK, 16), lambda i: (i, 0)),
        ],
        out_specs=[
            pl.BlockSpec((BLK, HF), lambda i: (i, 0)),
            pl.BlockSpec((BLK, HF), lambda i: (i, 0)),
            pl.BlockSpec((BLK, 1), lambda i: (i, 0)),
        ],
        out_shape=[
            jax.ShapeDtypeStruct((N, HF), _f32),
            jax.ShapeDtypeStruct((N, HF), _f32),
            jax.ShapeDtypeStruct((N, 1), _f32),
        ],
    )(x, W1, degA, degB)


def _tcl_body(aL_ref, aR_ref, dinv_ref, b_ref, w_ref, hL_ref, hR_ref):
    dinv = dinv_ref[...]
    agg = jnp.concatenate([aL_ref[...], aR_ref[...]], axis=1)
    t = jnp.maximum(agg * dinv + b_ref[...], 0.0)
    h = _dot(t, w_ref[...]) * dinv
    hL_ref[...] = h[:, :HF]
    hR_ref[...] = h[:, HF:]


def _tcl(aL, aR, dinv, b, W):
    return pl.pallas_call(
        _tcl_body,
        grid=(N // BLK,),
        in_specs=[
            pl.BlockSpec((BLK, HF), lambda i: (i, 0)),
            pl.BlockSpec((BLK, HF), lambda i: (i, 0)),
            pl.BlockSpec((BLK, 1), lambda i: (i, 0)),
            pl.BlockSpec((1, F), lambda i: (0, 0)),
            pl.BlockSpec((F, F), lambda i: (0, 0)),
        ],
        out_specs=[
            pl.BlockSpec((BLK, HF), lambda i: (i, 0)),
            pl.BlockSpec((BLK, HF), lambda i: (i, 0)),
        ],
        out_shape=[
            jax.ShapeDtypeStruct((N, HF), _f32),
            jax.ShapeDtypeStruct((N, HF), _f32),
        ],
    )(aL, aR, dinv, b, W)


def _tcf_body(aL_ref, aR_ref, dinv_ref, b_ref, wl_ref, bl_ref, y_ref):
    dinv = dinv_ref[...]
    agg = jnp.concatenate([aL_ref[...], aR_ref[...]], axis=1)
    t = jnp.maximum(agg * dinv + b_ref[...], 0.0)
    y = _dot(t, wl_ref[...]) + bl_ref[...]
    y_ref[...] = jax.nn.sigmoid(y)


def _tcf(aL, aR, dinv, b3, Wl, bl):
    return pl.pallas_call(
        _tcf_body,
        grid=(N // BLK,),
        in_specs=[
            pl.BlockSpec((BLK, HF), lambda i: (i, 0)),
            pl.BlockSpec((BLK, HF), lambda i: (i, 0)),
            pl.BlockSpec((BLK, 1), lambda i: (i, 0)),
            pl.BlockSpec((1, F), lambda i: (0, 0)),
            pl.BlockSpec((F, 1), lambda i: (0, 0)),
            pl.BlockSpec((1, 1), lambda i: (0, 0)),
        ],
        out_specs=[pl.BlockSpec((BLK, 1), lambda i: (i, 0))],
        out_shape=[jax.ShapeDtypeStruct((N, 1), _f32)],
    )(aL, aR, dinv, b3, Wl, bl)


def kernel(x, edge_index, W1, b1, W2, b2, W3, b3, Wl, bl):
    ones16 = jnp.ones((N, 16), _f32)
    src, dst = edge_index[0], edge_index[1]
    degA, degB = _sc_degree(dst, ones16)
    hL, hR, dinv = _tc1(x, W1, degA, degB)
    aL, aR = _sc_agg(src, dst, hL, hR)
    hL, hR = _tcl(aL, aR, dinv, b1.reshape(1, F), W2)
    aL, aR = _sc_agg(src, dst, hL, hR)
    hL, hR = _tcl(aL, aR, dinv, b2.reshape(1, F), W3)
    aL, aR = _sc_agg(src, dst, hL, hR)
    (y,) = _tcf(aL, aR, dinv, b3.reshape(1, F), Wl, bl.reshape(1, 1))
    return y.reshape(-1)


# speedup vs baseline: 14.7239x; 1.0036x over previous
import functools

import jax
import jax.numpy as jnp
from jax import lax
from jax.experimental import pallas as pl
from jax.experimental.pallas import tpu as pltpu
from jax.experimental.pallas import tpu_sc as plsc

N = 10000
E = 160000
F = 256
HF = 128
BLK = 1000
CH = 128
NCH = E // CH
NSUB = 16
RPS = 624
TAIL = N - RPS * NSUB

_f32 = jnp.float32
_mesh = plsc.VectorSubcoreMesh(core_axis_name="c", subcore_axis_name="s")


@functools.partial(
    pl.kernel,
    out_type=(jax.ShapeDtypeStruct((N, 16), _f32),
              jax.ShapeDtypeStruct((N, 16), _f32)),
    mesh=_mesh,
    scratch_types=[
        pltpu.VMEM((CH,), jnp.int32),
        pltpu.VMEM((CH, 16), _f32),
        pltpu.VMEM_SHARED((N, 16), _f32),
    ],
)
def _sc_degree(dst_hbm, ones_hbm, degA_hbm, degB_hbm, dstbuf, onesbuf, acc):
    c = lax.axis_index("c")
    s = lax.axis_index("s")

    def rowcopy(from_ref, to_ref):
        sl = pl.ds(s * RPS, RPS)
        pltpu.sync_copy(from_ref.at[sl], to_ref.at[sl])

        @pl.when(s == NSUB - 1)
        def _():
            tl = pl.ds(RPS * NSUB, TAIL)
            pltpu.sync_copy(from_ref.at[tl], to_ref.at[tl])

    rowcopy(ones_hbm, acc)
    pltpu.sync_copy(ones_hbm.at[pl.ds(0, CH)], onesbuf)
    plsc.subcore_barrier()
    w = c * NSUB + s
    lo = (w * NCH) // (2 * NSUB)
    hi = ((w + 1) * NCH) // (2 * NSUB)

    def chunk(j, carry):
        pltpu.sync_copy(dst_hbm.at[pl.ds(j * CH, CH)], dstbuf)
        pltpu.sync_copy(onesbuf, acc.at[dstbuf], add=True)
        return carry

    lax.fori_loop(lo, hi, chunk, 0)
    plsc.subcore_barrier()

    @pl.when(c == 0)
    def _():
        rowcopy(acc, degA_hbm)

    @pl.when(c == 1)
    def _():
        rowcopy(acc, degB_hbm)


EPS = E // NSUB
NCHS = EPS // CH
REM = EPS - NCHS * CH


@functools.partial(
    pl.kernel,
    out_type=(jax.ShapeDtypeStruct((N, HF), _f32),
              jax.ShapeDtypeStruct((N, HF), _f32)),
    mesh=_mesh,
    scratch_types=[
        pltpu.VMEM((EPS,), jnp.int32),
        pltpu.VMEM((2, CH), jnp.int32),
        pltpu.VMEM((1, REM), jnp.int32),
        pltpu.VMEM((2, CH, HF), _f32),
        pltpu.VMEM((REM, HF), _f32),
        pltpu.VMEM_SHARED((N, HF), _f32),
        pltpu.SemaphoreType.DMA,
        pltpu.SemaphoreType.DMA,
        pltpu.SemaphoreType.DMA,
        pltpu.SemaphoreType.DMA,
        pltpu.SemaphoreType.DMA,
        pltpu.SemaphoreType.DMA,
    ],
)
def _sc_agg(src_hbm, dst_hbm, hL_hbm, hR_hbm, aggL_hbm, aggR_hbm,
            srcslab, dstbuf, dstbuf16, rowbuf, rowbuf16, acc,
            semd0, semd1, semg0, semg1, sems0, sems1):
    c = lax.axis_index("c")
    s = lax.axis_index("s")
    semd = (semd0, semd1)
    semg = (semg0, semg1)
    sems = (sems0, sems1)

    def rowcopy(from_ref, to_ref):
        sl = pl.ds(s * RPS, RPS)
        pltpu.sync_copy(from_ref.at[sl], to_ref.at[sl])

        @pl.when(s == NSUB - 1)
        def _():
            tl = pl.ds(RPS * NSUB, TAIL)
            pltpu.sync_copy(from_ref.at[tl], to_ref.at[tl])

    def run(h_hbm, out_hbm):
        pltpu.sync_copy(src_hbm.at[pl.ds(s * EPS, EPS)], srcslab)
        rowcopy(h_hbm, acc)

        def dst_dma(j, b):
            return pltpu.make_async_copy(
                dst_hbm.at[pl.ds(s * EPS + j * CH, CH)], dstbuf.at[b], semd[b])

        def gat_dma(j, b):
            return pltpu.make_async_copy(
                h_hbm.at[srcslab.at[pl.ds(j * CH, CH)]], rowbuf.at[b], semg[b])

        def sct_dma(b):
            return pltpu.make_async_copy(rowbuf.at[b], acc.at[dstbuf.at[b]],
                                         sems[b])

        gat_dma(0, 0).start()
        dst_dma(0, 0).start()
        plsc.subcore_barrier()

        def pair(k, carry):
            for b in (0, 1):
                j = 2 * k + b
                gat_dma(j, b).wait()
                dst_dma(j, b).wait()
                sct_dma(b).start(add=True)

                @pl.when(j >= 1)
                def _():
                    sct_dma(1 - b).wait()

                @pl.when(j + 1 < NCHS)
                def _():
                    gat_dma(j + 1, 1 - b).start()
                    dst_dma(j + 1, 1 - b).start()

            return carry

        lax.fori_loop(0, NCHS // 2, pair, 0)
        sct_dma((NCHS - 1) % 2).wait()
        pltpu.sync_copy(dst_hbm.at[pl.ds(s * EPS + NCHS * CH, REM)],
                        dstbuf16.at[0])
        pltpu.async_copy(h_hbm.at[srcslab.at[pl.ds(NCHS * CH, REM)]],
                         rowbuf16, semg0).wait()
        pltpu.sync_copy(rowbuf16, acc.at[dstbuf16.at[0]], add=True)
        plsc.subcore_barrier()
        rowcopy(acc, out_hbm)

    @pl.when(c == 0)
    def _():
        run(hL_hbm, aggL_hbm)

    @pl.when(c == 1)
    def _():
        run(hR_hbm, aggR_hbm)


def _dot(a, b):
    return jnp.dot(a, b, preferred_element_type=_f32,
                   precision=lax.Precision.HIGHEST)


def _tc1_body(x_ref, w_ref, dA_ref, dB_ref, hL_ref, hR_ref, dinv_ref):
    deg = dA_ref[:, 0:1] + dB_ref[:, 0:1] - 1.0
    dinv = lax.rsqrt(deg)
    h = _dot(x_ref[...], w_ref[...]) * dinv
    hL_ref[...] = h[:, :HF]
    hR_ref[...] = h[:, HF:]
    dinv_ref[...] = dinv


def _tc1(x, W1, degA, degB):
    return pl.pallas_call(
        _tc1_body,
        grid=(N // BLK,),
        in_specs=[
            pl.BlockSpec((BLK, F), lambda i: (i, 0)),
            pl.BlockSpec((F, F), lambda i: (0, 0)),
            pl.BlockSpec((BLK, 16), lambda i: (i, 0)),
            pl.BlockSpec((BLK, 16), lambda i: (i, 0)),
        ],
        out_specs=[
            pl.BlockSpec((BLK, HF), lambda i: (i, 0)),
            pl.BlockSpec((BLK, HF), lambda i: (i, 0)),
            pl.BlockSpec((BLK, 1), lambda i: (i, 0)),
        ],
        out_shape=[
            jax.ShapeDtypeStruct((N, HF), _f32),
            jax.ShapeDtypeStruct((N, HF), _f32),
            jax.ShapeDtypeStruct((N, 1), _f32),
        ],
    )(x, W1, degA, degB)


def _tcl_body(aL_ref, aR_ref, dinv_ref, b_ref, w_ref, hL_ref, hR_ref):
    dinv = dinv_ref[...]
    agg = jnp.concatenate([aL_ref[...], aR_ref[...]], axis=1)
    t = jnp.maximum(agg * dinv + b_ref[...], 0.0)
    h = _dot(t, w_ref[...]) * dinv
    hL_ref[...] = h[:, :HF]
    hR_ref[...] = h[:, HF:]


def _tcl(aL, aR, dinv, b, W):
    return pl.pallas_call(
        _tcl_body,
        grid=(N // BLK,),
        in_specs=[
            pl.BlockSpec((BLK, HF), lambda i: (i, 0)),
            pl.BlockSpec((BLK, HF), lambda i: (i, 0)),
            pl.BlockSpec((BLK, 1), lambda i: (i, 0)),
            pl.BlockSpec((1, F), lambda i: (0, 0)),
            pl.BlockSpec((F, F), lambda i: (0, 0)),
        ],
        out_specs=[
            pl.BlockSpec((BLK, HF), lambda i: (i, 0)),
            pl.BlockSpec((BLK, HF), lambda i: (i, 0)),
        ],
        out_shape=[
            jax.ShapeDtypeStruct((N, HF), _f32),
            jax.ShapeDtypeStruct((N, HF), _f32),
        ],
    )(aL, aR, dinv, b, W)


def _tcf_body(aL_ref, aR_ref, dinv_ref, b_ref, wl_ref, bl_ref, y_ref):
    dinv = dinv_ref[...]
    agg = jnp.concatenate([aL_ref[...], aR_ref[...]], axis=1)
    t = jnp.maximum(agg * dinv + b_ref[...], 0.0)
    y = _dot(t, wl_ref[...]) + bl_ref[...]
    y_ref[...] = jax.nn.sigmoid(y)


def _tcf(aL, aR, dinv, b3, Wl, bl):
    return pl.pallas_call(
        _tcf_body,
        grid=(N // BLK,),
        in_specs=[
            pl.BlockSpec((BLK, HF), lambda i: (i, 0)),
            pl.BlockSpec((BLK, HF), lambda i: (i, 0)),
            pl.BlockSpec((BLK, 1), lambda i: (i, 0)),
            pl.BlockSpec((1, F), lambda i: (0, 0)),
            pl.BlockSpec((F, 1), lambda i: (0, 0)),
            pl.BlockSpec((1, 1), lambda i: (0, 0)),
        ],
        out_specs=[pl.BlockSpec((BLK, 1), lambda i: (i, 0))],
        out_shape=[jax.ShapeDtypeStruct((N, 1), _f32)],
    )(aL, aR, dinv, b3, Wl, bl)


def kernel(x, edge_index, W1, b1, W2, b2, W3, b3, Wl, bl):
    ones16 = jnp.ones((N, 16), _f32)
    src, dst = edge_index[0], edge_index[1]
    degA, degB = _sc_degree(dst, ones16)
    hL, hR, dinv = _tc1(x, W1, degA, degB)
    aL, aR = _sc_agg(src, dst, hL, hR)
    hL, hR = _tcl(aL, aR, dinv, b1.reshape(1, F), W2)
    aL, aR = _sc_agg(src, dst, hL, hR)
    hL, hR = _tcl(aL, aR, dinv, b2.reshape(1, F), W3)
    aL, aR = _sc_agg(src, dst, hL, hR)
    (y,) = _tcf(aL, aR, dinv, b3.reshape(1, F), Wl, bl.reshape(1, 1))
    return y.reshape(-1)
